# baseline (speedup 1.0000x reference)
import functools

import jax
import jax.numpy as jnp
from jax import lax
from jax.experimental import pallas as pl
from jax.experimental.pallas import tpu as pltpu
from jax.experimental.pallas import tpu_sc as plsc

_NUM_CORES = 2
_NUM_SUBCORES = 16
_NW = _NUM_CORES * _NUM_SUBCORES
_LANES = 16
_R = 16
_B_SC = 3
_TC_ROWS = 512


@functools.lru_cache(maxsize=None)
def _build_sc(B, T, DIM, b_sc):
    rows_w = T // _NW
    R = _R if rows_w % _R == 0 else rows_w
    n_sub = rows_w // R
    n_tiles = n_sub * b_sc
    groups_row = DIM // _LANES

    mesh = plsc.VectorSubcoreMesh(core_axis_name="c", subcore_axis_name="s")

    @functools.partial(
        pl.kernel,
        out_type=jax.ShapeDtypeStruct((b_sc, T, DIM), jnp.float32),
        mesh=mesh,
        compiler_params=pltpu.CompilerParams(use_tc_tiling_on_sc=True),
        scratch_types=(
            [pltpu.VMEM((R, DIM), jnp.float32) for _ in range(4)]
            + [pltpu.VMEM((R, DIM), jnp.float32) for _ in range(2)]
            + [pltpu.SemaphoreType.DMA for _ in range(10)]
        ),
    )
    def k(x_hbm, pos_hbm, out_hbm,
          xv0, xv1, xv2, xv3, pv0, pv1,
          sxi0, sxi1, sxi2, sxi3, soo0, soo1, soo2, soo3, spi0, spi1):
        wid = lax.axis_index("s") * _NUM_CORES + lax.axis_index("c")
        base = wid * rows_w
        xv = (xv0, xv1, xv2, xv3)
        pv = (pv0, pv1)
        sxi = (sxi0, sxi1, sxi2, sxi3)
        soo = (soo0, soo1, soo2, soo3)
        spi = (spi0, spi1)

        def x_loc(kk):
            s, b = divmod(kk, b_sc)
            return b, base + s * R

        def start_xin(kk):
            b, r0 = x_loc(kk)
            return pltpu.async_copy(
                x_hbm.at[b, pl.ds(r0, R), :], xv[kk % 4], sxi[kk % 4])

        def start_pin(s):
            return pltpu.async_copy(
                pos_hbm.at[pl.ds(base + s * R, R), :], pv[s % 2], spi[s % 2])

        def start_out(kk):
            b, r0 = x_loc(kk)
            return pltpu.async_copy(
                xv[kk % 4], out_hbm.at[b, pl.ds(r0, R), :], soo[kk % 4])

        pending = {}
        pending["p0"] = start_pin(0)
        for j in range(min(4, n_tiles)):
            pending[f"x{j}"] = start_xin(j)

        for kk in range(n_tiles):
            s, b = divmod(kk, b_sc)
            if b == 0:
                pending.pop(f"p{s}").wait()
                if s + 1 < n_sub:
                    pending[f"p{s + 1}"] = start_pin(s + 1)
            pending.pop(f"x{kk}").wait()

            xbuf = xv[kk % 4]
            pbuf = pv[s % 2]

            @plsc.parallel_loop(0, R * groups_row, step=1, unroll=8)
            def add_body(i):
                r = i // groups_row
                sl = pl.ds((i % groups_row) * _LANES, _LANES)
                xbuf[r, sl] = xbuf[r, sl] + pbuf[r, sl]

            pending[f"o{kk}"] = start_out(kk)
            if kk >= 2 and kk + 2 < n_tiles:
                pending.pop(f"o{kk - 2}").wait()
                pending[f"x{kk + 2}"] = start_xin(kk + 2)

        for h in pending.values():
            h.wait()

    return k


def _tc_body(x_ref, pos_ref, out_ref):
    out_ref[...] = x_ref[...] + pos_ref[...]


@functools.lru_cache(maxsize=None)
def _build_tc(B, T, DIM, b_sc):
    b_tc = B - b_sc
    rows = _TC_ROWS if T % _TC_ROWS == 0 else T
    grid = (b_tc, T // rows)
    return pl.pallas_call(
        _tc_body,
        grid=grid,
        in_specs=[
            pl.BlockSpec((1, rows, DIM), lambda b, t: (b + b_sc, t, 0)),
            pl.BlockSpec((rows, DIM), lambda b, t: (t, 0)),
        ],
        out_specs=pl.BlockSpec((1, rows, DIM), lambda b, t: (b, t, 0)),
        out_shape=jax.ShapeDtypeStruct((b_tc, T, DIM), jnp.float32),
        compiler_params=pltpu.CompilerParams(
            dimension_semantics=("parallel", "parallel")),
    )


def kernel(x, pos):
    B, T, DIM = x.shape
    pos = pos[:T]
    b_sc = min(_B_SC, B)
    out_sc = _build_sc(B, T, DIM, b_sc)(x, pos)
    if b_sc == B:
        return out_sc
    out_tc = _build_tc(B, T, DIM, b_sc)(x, pos)
    return jnp.concatenate([out_sc, out_tc], axis=0)


# speedup vs baseline: 1.4265x; 1.4265x over previous
import functools

import jax
import jax.numpy as jnp
from jax import lax
from jax.experimental import pallas as pl
from jax.experimental.pallas import tpu as pltpu
from jax.experimental.pallas import tpu_sc as plsc

_NUM_CORES = 2
_NUM_SUBCORES = 16
_NW = _NUM_CORES * _NUM_SUBCORES
_LANES = 16
_R = 16


@functools.lru_cache(maxsize=None)
def _build(B, T, DIM):
    rows_w = T // _NW
    R = _R if rows_w % _R == 0 else rows_w
    n_sub = rows_w // R
    n_tiles = n_sub * B
    groups_row = DIM // _LANES

    mesh = plsc.VectorSubcoreMesh(core_axis_name="c", subcore_axis_name="s")

    @functools.partial(
        pl.kernel,
        out_type=jax.ShapeDtypeStruct((B, T, DIM), jnp.float32),
        mesh=mesh,
        compiler_params=pltpu.CompilerParams(use_tc_tiling_on_sc=True),
        scratch_types=(
            [pltpu.VMEM((R, DIM), jnp.float32) for _ in range(4)]
            + [pltpu.VMEM((R, DIM), jnp.float32) for _ in range(2)]
            + [pltpu.SemaphoreType.DMA for _ in range(10)]
        ),
    )
    def k(x_hbm, pos_hbm, out_hbm,
          xv0, xv1, xv2, xv3, pv0, pv1,
          sxi0, sxi1, sxi2, sxi3, soo0, soo1, soo2, soo3, spi0, spi1):
        wid = lax.axis_index("s") * _NUM_CORES + lax.axis_index("c")
        base = wid * rows_w
        xv = (xv0, xv1, xv2, xv3)
        pv = (pv0, pv1)
        sxi = (sxi0, sxi1, sxi2, sxi3)
        soo = (soo0, soo1, soo2, soo3)
        spi = (spi0, spi1)

        def x_loc(kk):
            s, b = divmod(kk, B)
            return b, base + s * R

        def start_xin(kk):
            b, r0 = x_loc(kk)
            return pltpu.async_copy(
                x_hbm.at[b, pl.ds(r0, R), :], xv[kk % 4], sxi[kk % 4])

        def start_pin(s):
            return pltpu.async_copy(
                pos_hbm.at[pl.ds(base + s * R, R), :], pv[s % 2], spi[s % 2])

        def start_out(kk):
            b, r0 = x_loc(kk)
            return pltpu.async_copy(
                xv[kk % 4], out_hbm.at[b, pl.ds(r0, R), :], soo[kk % 4])

        pending = {}
        pending["p0"] = start_pin(0)
        for j in range(min(4, n_tiles)):
            pending[f"x{j}"] = start_xin(j)

        for kk in range(n_tiles):
            s, b = divmod(kk, B)
            if b == 0:
                pending.pop(f"p{s}").wait()
                if s + 1 < n_sub:
                    pending[f"p{s + 1}"] = start_pin(s + 1)
            pending.pop(f"x{kk}").wait()

            xbuf = xv[kk % 4]
            pbuf = pv[s % 2]

            @plsc.parallel_loop(0, R * groups_row, step=1, unroll=16)
            def add_body(i):
                r = i // groups_row
                sl = pl.ds((i % groups_row) * _LANES, _LANES)
                xbuf[r, sl] = xbuf[r, sl] + pbuf[r, sl]

            pending[f"o{kk}"] = start_out(kk)
            if kk >= 2 and kk + 2 < n_tiles:
                pending.pop(f"o{kk - 2}").wait()
                pending[f"x{kk + 2}"] = start_xin(kk + 2)

        for h in pending.values():
            h.wait()

    return k


def kernel(x, pos):
    B, T, DIM = x.shape
    return _build(B, T, DIM)(x, pos[:T])


# speedup vs baseline: 1.4735x; 1.0329x over previous
import functools

import jax
import jax.numpy as jnp
from jax import lax
from jax.experimental import pallas as pl
from jax.experimental.pallas import tpu as pltpu
from jax.experimental.pallas import tpu_sc as plsc

_NUM_CORES = 2
_NUM_SUBCORES = 16
_NW = _NUM_CORES * _NUM_SUBCORES
_LANES = 16
_R = 16


@functools.lru_cache(maxsize=None)
def _build(B, T, DIM):
    rows_w = T // _NW
    R = _R if rows_w % _R == 0 else rows_w
    n_sub = rows_w // R
    n_tiles = n_sub * B
    groups_row = DIM // _LANES

    mesh = plsc.VectorSubcoreMesh(core_axis_name="c", subcore_axis_name="s")

    @functools.partial(
        pl.kernel,
        out_type=jax.ShapeDtypeStruct((B, T, DIM), jnp.float32),
        mesh=mesh,
        compiler_params=pltpu.CompilerParams(use_tc_tiling_on_sc=True),
        scratch_types=(
            [pltpu.VMEM((R, DIM), jnp.float32) for _ in range(4)]
            + [pltpu.VMEM((R, DIM), jnp.float32) for _ in range(2)]
            + [pltpu.SemaphoreType.DMA for _ in range(10)]
        ),
    )
    def k(x_hbm, pos_hbm, out_hbm,
          xv0, xv1, xv2, xv3, pv0, pv1,
          sxi0, sxi1, sxi2, sxi3, soo0, soo1, soo2, soo3, spi0, spi1):
        wid = lax.axis_index("s") * _NUM_CORES + lax.axis_index("c")
        base = wid * rows_w
        xv = (xv0, xv1, xv2, xv3)
        pv = (pv0, pv1)
        sxi = (sxi0, sxi1, sxi2, sxi3)
        soo = (soo0, soo1, soo2, soo3)
        spi = (spi0, spi1)

        def x_loc(kk):
            s, b = divmod(kk, B)
            return b, base + s * R

        def start_xin(kk):
            b, r0 = x_loc(kk)
            return pltpu.async_copy(
                x_hbm.at[b, pl.ds(r0, R), :], xv[kk % 4], sxi[kk % 4])

        def start_pin(s):
            return pltpu.async_copy(
                pos_hbm.at[pl.ds(base + s * R, R), :], pv[s % 2], spi[s % 2])

        def start_out(kk):
            b, r0 = x_loc(kk)
            return pltpu.async_copy(
                xv[kk % 4], out_hbm.at[b, pl.ds(r0, R), :], soo[kk % 4])

        pending = {}
        pending["p0"] = start_pin(0)
        for j in range(min(4, n_tiles)):
            pending[f"x{j}"] = start_xin(j)

        for kk in range(n_tiles):
            s, b = divmod(kk, B)
            if b == 0:
                pending.pop(f"p{s}").wait()
                if s + 1 < n_sub:
                    pending[f"p{s + 1}"] = start_pin(s + 1)
            pending.pop(f"x{kk}").wait()

            xbuf = xv[kk % 4]
            pbuf = pv[s % 2]

            @plsc.parallel_loop(0, R * groups_row, step=1, unroll=8)
            def add_body(i):
                r = i // groups_row
                sl = pl.ds((i % groups_row) * _LANES, _LANES)
                plsc.addupdate(xbuf.at[r, sl], pbuf[r, sl])

            pending[f"o{kk}"] = start_out(kk)
            if kk >= 2 and kk + 2 < n_tiles:
                pending.pop(f"o{kk - 2}").wait()
                pending[f"x{kk + 2}"] = start_xin(kk + 2)

        for h in pending.values():
            h.wait()

    return k


def kernel(x, pos):
    B, T, DIM = x.shape
    return _build(B, T, DIM)(x, pos[:T])


# speedup vs baseline: 1.5631x; 1.0608x over previous
import functools

import jax
import jax.numpy as jnp
from jax import lax
from jax.experimental import pallas as pl
from jax.experimental.pallas import tpu as pltpu
from jax.experimental.pallas import tpu_sc as plsc

_NUM_CORES = 2
_NUM_SUBCORES = 16
_NW = _NUM_CORES * _NUM_SUBCORES
_LANES = 16
_R = 16


@functools.lru_cache(maxsize=None)
def _build(B, T, DIM):
    rows_w = T // _NW
    R = _R if rows_w % _R == 0 else rows_w
    n_sub = rows_w // R
    n_tiles = n_sub * B
    groups_row = DIM // _LANES

    mesh = plsc.VectorSubcoreMesh(core_axis_name="c", subcore_axis_name="s")

    @functools.partial(
        pl.kernel,
        out_type=jax.ShapeDtypeStruct((B, T, DIM), jnp.float32),
        mesh=mesh,
        compiler_params=pltpu.CompilerParams(use_tc_tiling_on_sc=True),
        scratch_types=(
            [pltpu.VMEM((R, DIM), jnp.float32) for _ in range(4)]
            + [pltpu.VMEM((R, DIM), jnp.float32) for _ in range(2)]
            + [pltpu.SemaphoreType.DMA for _ in range(10)]
        ),
    )
    def k(x_hbm, pos_hbm, out_hbm,
          xv0, xv1, xv2, xv3, pv0, pv1,
          sxi0, sxi1, sxi2, sxi3, soo0, soo1, soo2, soo3, spi0, spi1):
        wid = lax.axis_index("s") * _NUM_CORES + lax.axis_index("c")
        base = wid * rows_w
        xv = (xv0, xv1, xv2, xv3)
        pv = (pv0, pv1)
        sxi = (sxi0, sxi1, sxi2, sxi3)
        soo = (soo0, soo1, soo2, soo3)
        spi = (spi0, spi1)

        def x_loc(kk):
            s, b = divmod(kk, B)
            return b, base + s * R

        def start_xin(kk):
            b, r0 = x_loc(kk)
            return pltpu.async_copy(
                x_hbm.at[b, pl.ds(r0, R), :], xv[kk % 4], sxi[kk % 4])

        def start_pin(s):
            return pltpu.async_copy(
                pos_hbm.at[pl.ds(base + s * R, R), :], pv[s % 2], spi[s % 2])

        def start_out(kk):
            b, r0 = x_loc(kk)
            return pltpu.async_copy(
                xv[kk % 4], out_hbm.at[b, pl.ds(r0, R), :], soo[kk % 4])

        pending = {}
        pending["p0"] = start_pin(0)
        for j in range(min(4, n_tiles)):
            pending[f"x{j}"] = start_xin(j)

        for st in range(n_tiles // 2):
            kk0 = 2 * st
            kk1 = kk0 + 1
            s, b0 = divmod(kk0, B)
            if b0 == 0:
                pending.pop(f"p{s}").wait()
                if s + 1 < n_sub:
                    pending[f"p{s + 1}"] = start_pin(s + 1)
            if st >= 1 and kk0 + 2 < n_tiles:
                pending.pop(f"o{kk0 - 2}").wait()
                pending[f"x{kk0 + 2}"] = start_xin(kk0 + 2)
                pending.pop(f"o{kk1 - 2}").wait()
                pending[f"x{kk1 + 2}"] = start_xin(kk1 + 2)
            pending.pop(f"x{kk0}").wait()
            pending.pop(f"x{kk1}").wait()

            xb0 = xv[kk0 % 4]
            xb1 = xv[kk1 % 4]
            pbuf = pv[s % 2]

            @plsc.parallel_loop(0, R * groups_row, step=1, unroll=8)
            def add_body(i):
                r = i // groups_row
                sl = pl.ds((i % groups_row) * _LANES, _LANES)
                pval = pbuf[r, sl]
                plsc.addupdate(xb0.at[r, sl], pval)
                plsc.addupdate(xb1.at[r, sl], pval)

            pending[f"o{kk0}"] = start_out(kk0)
            pending[f"o{kk1}"] = start_out(kk1)

        for h in pending.values():
            h.wait()

    return k


def kernel(x, pos):
    B, T, DIM = x.shape
    return _build(B, T, DIM)(x, pos[:T])
